# Initial kernel scaffold; baseline (speedup 1.0000x reference)
import jax
import jax.numpy as jnp
from jax.experimental import pallas as pl


def kernel(scores, bbox_deltas, im_info, anchors):
    raise NotImplementedError("write your pallas kernel here")


# speedup vs baseline: 6.5978x; 6.5978x over previous
import functools

import jax
import jax.numpy as jnp
from jax.experimental import pallas as pl
from jax.experimental.pallas import tpu as pltpu

_FEAT_STRIDE = 16.0
_PRE_NMS = 6000
_POST_NMS = 300
_NMS_THRESH = 0.7
_LANES = 128


def _proposal_kernel(im_ref, sb_ref, dx_ref, dy_ref, dw_ref, dh_ref,
                     cx_ref, cy_ref, aw_ref, ah_ref, out_ref,
                     x1_s, y1_s, x2_s, y2_s, ar_s, sm_s,
                     *, rows, pre_nms, post_nms, thresh):
    b = pl.program_id(0)
    n = rows * _LANES

    ww = aw_ref[...]
    hh = ah_ref[...]
    pcx = dx_ref[0] * ww + cx_ref[...]
    pcy = dy_ref[0] * hh + cy_ref[...]
    pw = jnp.exp(dw_ref[0]) * ww
    ph = jnp.exp(dh_ref[0]) * hh
    maxw = im_ref[b, 1] - 1.0
    maxh = im_ref[b, 0] - 1.0
    x1 = jnp.minimum(jnp.maximum(pcx - 0.5 * pw, 0.0), maxw)
    y1 = jnp.minimum(jnp.maximum(pcy - 0.5 * ph, 0.0), maxh)
    x2 = jnp.minimum(jnp.maximum(pcx + 0.5 * pw, 0.0), maxw)
    y2 = jnp.minimum(jnp.maximum(pcy + 0.5 * ph, 0.0), maxh)
    x1_s[...] = x1
    y1_s[...] = y1
    x2_s[...] = x2
    y2_s[...] = y2
    ar_s[...] = (x2 - x1 + 1.0) * (y2 - y1 + 1.0)

    sbits = sb_ref[0]
    iota = (jax.lax.broadcasted_iota(jnp.int32, (rows, _LANES), 0) * _LANES
            + jax.lax.broadcasted_iota(jnp.int32, (rows, _LANES), 1))

    def _count_gt(t):
        return jnp.sum(jnp.where(sbits > t, 1.0, 0.0))

    kf = jnp.float32(pre_nms)

    def _bis_body(_, carry):
        lo, hi = carry
        mid = jax.lax.div(lo + hi, 2)
        gt = _count_gt(mid) >= kf
        return (jnp.where(gt, mid, lo), jnp.where(gt, hi, mid))

    lo0 = jnp.int32(-1)
    hi0 = jnp.int32(1 << 30)
    _, tbits = jax.lax.fori_loop(0, 31, _bis_body, (lo0, hi0))

    count_gt = _count_gt(tbits)
    r = kf - count_gt
    eq = sbits == tbits

    def _tie_body(_, carry):
        lo, hi = carry
        mid = jax.lax.div(lo + hi, 2)
        cnt = jnp.sum(jnp.where(eq & (iota <= mid), 1.0, 0.0))
        ge = cnt >= r
        return (jnp.where(ge, lo, mid), jnp.where(ge, mid, hi))

    _, mcut = jax.lax.fori_loop(0, 18, _tie_body, (jnp.int32(-1),
                                                   jnp.int32(n - 1)))

    member = (sbits > tbits) | (eq & (iota <= mcut))
    sm_s[...] = jnp.where(member, sbits, jnp.int32(-1))

    def _nms_body(j, carry):
        smv = sm_s[...]
        mv = jnp.max(smv)
        sel = smv == mv
        idxv = jnp.min(jnp.where(sel, iota, jnp.int32(n)))
        one = iota == idxv
        bx1 = jnp.sum(jnp.where(one, x1_s[...], 0.0))
        by1 = jnp.sum(jnp.where(one, y1_s[...], 0.0))
        bx2 = jnp.sum(jnp.where(one, x2_s[...], 0.0))
        by2 = jnp.sum(jnp.where(one, y2_s[...], 0.0))
        bar = jnp.sum(jnp.where(one, ar_s[...], 0.0))
        xx1 = jnp.maximum(bx1, x1_s[...])
        yy1 = jnp.maximum(by1, y1_s[...])
        xx2 = jnp.minimum(bx2, x2_s[...])
        yy2 = jnp.minimum(by2, y2_s[...])
        iw = jnp.maximum(0.0, xx2 - xx1 + 1.0)
        ih = jnp.maximum(0.0, yy2 - yy1 + 1.0)
        inter = iw * ih
        iou = inter / ((bar + ar_s[...]) - inter)
        sm_s[...] = jnp.where(iou <= thresh, smv, jnp.int32(-1))
        vf = jnp.where(mv >= 0, 1.0, 0.0)
        out_ref[0, j, 0] = bx1 * vf
        out_ref[0, j, 1] = by1 * vf
        out_ref[0, j, 2] = bx2 * vf
        out_ref[0, j, 3] = by2 * vf
        return carry

    jax.lax.fori_loop(0, post_nms, _nms_body, 0)


def kernel(scores, bbox_deltas, im_info, anchors):
    B = scores.shape[0]
    A = anchors.shape[0]
    H = scores.shape[2]
    W = scores.shape[3]
    K = H * W
    N = K * A
    rows = N // _LANES

    sc = jnp.transpose(scores[:, A:, :, :], (0, 2, 3, 1)).reshape(B, rows,
                                                                  _LANES)
    sbits = jax.lax.bitcast_convert_type(sc, jnp.int32)
    dl = jnp.transpose(bbox_deltas, (0, 2, 3, 1)).reshape(B, K, A, 4)
    dx = dl[..., 0].reshape(B, rows, _LANES)
    dy = dl[..., 1].reshape(B, rows, _LANES)
    dw = dl[..., 2].reshape(B, rows, _LANES)
    dh = dl[..., 3].reshape(B, rows, _LANES)

    aw = anchors[:, 2] - anchors[:, 0] + 1.0
    ah = anchors[:, 3] - anchors[:, 1] + 1.0
    acx = anchors[:, 0] + 0.5 * aw
    acy = anchors[:, 1] + 0.5 * ah
    shift_x = jnp.arange(W, dtype=jnp.float32) * _FEAT_STRIDE
    shift_y = jnp.arange(H, dtype=jnp.float32) * _FEAT_STRIDE
    sx, sy = jnp.meshgrid(shift_x, shift_y)
    cx = (sx.ravel()[:, None] + acx[None, :]).reshape(rows, _LANES)
    cy = (sy.ravel()[:, None] + acy[None, :]).reshape(rows, _LANES)
    awf = jnp.broadcast_to(aw[None, :], (K, A)).reshape(rows, _LANES)
    ahf = jnp.broadcast_to(ah[None, :], (K, A)).reshape(rows, _LANES)

    body = functools.partial(_proposal_kernel, rows=rows, pre_nms=_PRE_NMS,
                             post_nms=_POST_NMS, thresh=_NMS_THRESH)

    img_spec = pl.BlockSpec((1, rows, _LANES), lambda b: (b, 0, 0))
    shared_spec = pl.BlockSpec((rows, _LANES), lambda b: (0, 0))
    out = pl.pallas_call(
        body,
        grid=(B,),
        in_specs=[
            pl.BlockSpec(memory_space=pltpu.SMEM),
            img_spec, img_spec, img_spec, img_spec, img_spec,
            shared_spec, shared_spec, shared_spec, shared_spec,
        ],
        out_specs=pl.BlockSpec((1, _POST_NMS, 4), lambda b: (b, 0, 0),
                               memory_space=pltpu.SMEM),
        out_shape=jax.ShapeDtypeStruct((B, _POST_NMS, 4), jnp.float32),
        scratch_shapes=[pltpu.VMEM((rows, _LANES), jnp.float32)] * 5
        + [pltpu.VMEM((rows, _LANES), jnp.int32)],
    )(im_info, sbits, dx, dy, dw, dh, cx, cy, awf, ahf)

    col0 = jnp.broadcast_to(
        jnp.arange(B, dtype=jnp.float32)[:, None, None], (B, _POST_NMS, 1))
    return jnp.concatenate([col0, out], axis=2)
